# Initial kernel scaffold; baseline (speedup 1.0000x reference)
import jax
import jax.numpy as jnp
from jax.experimental import pallas as pl


def kernel(X, W, theta, freqs):
    raise NotImplementedError("write your pallas kernel here")


# speedup vs baseline: 1.6786x; 1.6786x over previous
import math

import jax
import jax.numpy as jnp
from jax.experimental import pallas as pl

_BIG = 1e30


def _sw_body(x_ref, w_ref, th_ref, L_ref, pif_ref, ipif_ref, out_ref):
    n, d_in = x_ref.shape[1], x_ref.shape[2]
    npad = w_ref.shape[1]
    m = th_ref.shape[1]

    x = x_ref[0]
    wcol = w_ref[0]
    proj = jnp.dot(x, th_ref[...], preferred_element_type=jnp.float32,
                   precision=jax.lax.Precision.HIGHEST)
    key = jnp.concatenate(
        [proj, jnp.full((npad - n, m), _BIG, jnp.float32)], axis=0)
    val = jnp.broadcast_to(wcol, (npad, m))

    idx = jax.lax.broadcasted_iota(jnp.int32, (npad, m), 0)
    k = 2
    while k <= npad:
        j = k // 2
        while j >= 1:
            bitset = (idx & j) != 0
            asc = (idx & k) == 0
            want_min = jnp.logical_xor(bitset, asc)
            pk = jnp.where(bitset, jnp.roll(key, j, axis=0),
                           jnp.roll(key, -j, axis=0))
            pv = jnp.where(bitset, jnp.roll(val, j, axis=0),
                           jnp.roll(val, -j, axis=0))
            le = key <= pk
            take_self = jnp.logical_or(jnp.logical_and(want_min, le),
                                       jnp.logical_and(~want_min, ~(key < pk)))
            key = jnp.where(take_self, key, pk)
            val = jnp.where(take_self, val, pv)
            j //= 2
        k *= 2

    c = jnp.dot(L_ref[...], val, preferred_element_type=jnp.float32,
                precision=jax.lax.Precision.HIGHEST)
    c_prev = c - val
    pif = pif_ref[...]
    coef = (jnp.sin(c * pif) - jnp.sin(c_prev * pif)) * ipif_ref[...]
    emb = math.sqrt(2.0) * jnp.sum(key * coef, axis=0, keepdims=True)
    out_ref[0] = emb


def kernel(X, W, theta, freqs):
    b, n, d_in = X.shape
    m = theta.shape[0]
    npad = 1 << (n - 1).bit_length()

    wn = W / jnp.sum(W, axis=-1, keepdims=True)
    wcol = jnp.pad(wn, ((0, 0), (0, npad - n)))[..., None]
    th_t = theta.T
    L = jnp.tril(jnp.ones((npad, npad), jnp.float32))
    pif = (jnp.pi * freqs)[None, :].astype(jnp.float32)
    ipif = (1.0 / (jnp.pi * freqs))[None, :].astype(jnp.float32)

    out = pl.pallas_call(
        _sw_body,
        grid=(b,),
        in_specs=[
            pl.BlockSpec((1, n, d_in), lambda i: (i, 0, 0)),
            pl.BlockSpec((1, npad, 1), lambda i: (i, 0, 0)),
            pl.BlockSpec((d_in, m), lambda i: (0, 0)),
            pl.BlockSpec((npad, npad), lambda i: (0, 0)),
            pl.BlockSpec((1, m), lambda i: (0, 0)),
            pl.BlockSpec((1, m), lambda i: (0, 0)),
        ],
        out_specs=pl.BlockSpec((1, 1, m), lambda i: (i, 0, 0)),
        out_shape=jax.ShapeDtypeStruct((b, 1, m), jnp.float32),
    )(X, wcol, th_t, L, pif, ipif)
    return out[:, 0, :]


# speedup vs baseline: 2.0470x; 1.2195x over previous
import math

import jax
import jax.numpy as jnp
from jax.experimental import pallas as pl

_BIG = 1e30
_BB = 4


def _sw_body(x_ref, w_ref, th_ref, L_ref, sel_ref, pif_ref, ipif_ref,
             out_ref):
    n, d_in = x_ref.shape[0] // _BB, x_ref.shape[1]
    npad = w_ref.shape[0] // _BB
    m = th_ref.shape[1]
    s_tot = _BB * npad

    x = x_ref[...]
    proj = jnp.dot(x, th_ref[...], preferred_element_type=jnp.float32,
                   precision=jax.lax.Precision.HIGHEST)
    pad = jnp.full((npad - n, m), _BIG, jnp.float32)
    pieces = []
    for bb in range(_BB):
        pieces.append(proj[bb * n:(bb + 1) * n])
        pieces.append(pad)
    key = jnp.concatenate(pieces, axis=0)
    val = jnp.broadcast_to(w_ref[...], (s_tot, m))

    idx = jax.lax.broadcasted_iota(jnp.int32, (s_tot, m), 0) & (npad - 1)

    k = 2
    while k <= npad:
        j = k // 2
        while j >= 8:
            g = s_tot // (2 * j)
            k4 = key.reshape(g, 2, j, m)
            v4 = val.reshape(g, 2, j, m)
            ka, kb = k4[:, 0], k4[:, 1]
            va, vb = v4[:, 0], v4[:, 1]
            le = ka <= kb
            if k < npad:
                desc = (((jax.lax.broadcasted_iota(jnp.int32, (g, j, m), 0)
                          * (2 * j)) & k) != 0)
                sel = jnp.logical_xor(le, desc)
            else:
                sel = le
            k0 = jnp.where(sel, ka, kb)
            k1 = jnp.where(sel, kb, ka)
            v0 = jnp.where(sel, va, vb)
            v1 = jnp.where(sel, vb, va)
            key = jnp.stack([k0, k1], axis=1).reshape(s_tot, m)
            val = jnp.stack([v0, v1], axis=1).reshape(s_tot, m)
            j //= 2
        while j >= 1:
            bitset = (idx & j) != 0
            asc = (idx & k) == 0 if k < npad else None
            pk = jnp.where(bitset, jnp.roll(key, j, axis=0),
                           jnp.roll(key, -j, axis=0))
            pv = jnp.where(bitset, jnp.roll(val, j, axis=0),
                           jnp.roll(val, -j, axis=0))
            le = key <= pk
            ge = ~(key < pk)
            if asc is None:
                want_min = ~bitset
            else:
                want_min = jnp.logical_xor(bitset, asc)
            take_self = jnp.logical_or(jnp.logical_and(want_min, le),
                                       jnp.logical_and(~want_min, ge))
            key = jnp.where(take_self, key, pk)
            val = jnp.where(take_self, val, pv)
            j //= 2
        k *= 2

    L = L_ref[...]
    c = jnp.concatenate(
        [jnp.dot(L, val[bb * npad:(bb + 1) * npad],
                 preferred_element_type=jnp.float32,
                 precision=jax.lax.Precision.HIGHEST)
         for bb in range(_BB)], axis=0)

    key_m = jnp.where(idx < n, key, 0.0)
    nxt = jnp.roll(key_m, -1, axis=0)
    nxt = jnp.where(idx == npad - 1, 0.0, nxt)
    t = (key_m - nxt) * jnp.sin(c * pif_ref[...])
    acc = jnp.dot(sel_ref[...], t, preferred_element_type=jnp.float32,
                  precision=jax.lax.Precision.HIGHEST)
    out_ref[0] = math.sqrt(2.0) * acc * ipif_ref[...]


def kernel(X, W, theta, freqs):
    b, n, d_in = X.shape
    m = theta.shape[0]
    npad = 1 << (n - 1).bit_length()
    s_tot = _BB * npad

    wn = W / jnp.sum(W, axis=-1, keepdims=True)
    wcol = jnp.pad(wn, ((0, 0), (0, npad - n))).reshape(b * npad, 1)
    x2 = X.reshape(b * n, d_in)
    th_t = theta.T
    L = jnp.tril(jnp.ones((npad, npad), jnp.float32))
    seg = jnp.arange(s_tot, dtype=jnp.int32) // npad
    sel_m = (seg[None, :] == jnp.arange(_BB, dtype=jnp.int32)[:, None])
    sel_m = sel_m.astype(jnp.float32)
    pif = (jnp.pi * freqs)[None, :].astype(jnp.float32)
    ipif = (1.0 / (jnp.pi * freqs))[None, :].astype(jnp.float32)

    grid = b // _BB
    out = pl.pallas_call(
        _sw_body,
        grid=(grid,),
        in_specs=[
            pl.BlockSpec((_BB * n, d_in), lambda i: (i, 0)),
            pl.BlockSpec((s_tot, 1), lambda i: (i, 0)),
            pl.BlockSpec((d_in, m), lambda i: (0, 0)),
            pl.BlockSpec((npad, npad), lambda i: (0, 0)),
            pl.BlockSpec((_BB, s_tot), lambda i: (0, 0)),
            pl.BlockSpec((1, m), lambda i: (0, 0)),
            pl.BlockSpec((1, m), lambda i: (0, 0)),
        ],
        out_specs=pl.BlockSpec((1, _BB, m), lambda i: (i, 0, 0)),
        out_shape=jax.ShapeDtypeStruct((grid, _BB, m), jnp.float32),
    )(x2, wcol, th_t, L, sel_m, pif, ipif)
    return out.reshape(b, m)


# speedup vs baseline: 2.8844x; 1.4091x over previous
import math

import jax
import jax.numpy as jnp
import numpy as np
from jax.experimental import pallas as pl

_BIG = 1e30
_BB = 4


def _sigma(i):
    return ((i & 7) << 5) | (i & 24) | ((i >> 5) & 7)


def _sw_body(x_ref, w_ref, th_ref, LD_ref, sel_ref, pif_ref, ipif_ref,
             out_ref):
    npad = w_ref.shape[0] // _BB
    m = th_ref.shape[1]
    s_tot = _BB * npad
    n = 200

    idxp = jax.lax.broadcasted_iota(jnp.int32, (s_tot, m), 0) & (npad - 1)
    il = ((idxp & 7) << 5) | (idxp & 24) | ((idxp >> 5) & 7)

    proj = jnp.dot(x_ref[...], th_ref[...],
                   preferred_element_type=jnp.float32,
                   precision=jax.lax.Precision.HIGHEST)
    key = jnp.where(il < n, proj, _BIG)
    val = jnp.broadcast_to(w_ref[...], (s_tot, m))

    k = 2
    while k <= npad:
        kp = _sigma(k) if k < npad else 0
        j = k // 2
        while j >= 1:
            jp = _sigma(j)
            if jp >= 8:
                g = s_tot // (2 * jp)
                k4 = key.reshape(g, 2, jp, m)
                v4 = val.reshape(g, 2, jp, m)
                ka, kb = k4[:, 0], k4[:, 1]
                va, vb = v4[:, 0], v4[:, 1]
                le = ka <= kb
                if kp:
                    i4 = idxp.reshape(g, 2, jp, m)
                    desc = (i4[:, 0] & kp) != 0
                    sel = jnp.logical_xor(le, desc)
                else:
                    sel = le
                k0 = jnp.where(sel, ka, kb)
                k1 = jnp.where(sel, kb, ka)
                v0 = jnp.where(sel, va, vb)
                v1 = jnp.where(sel, vb, va)
                key = jnp.stack([k0, k1], axis=1).reshape(s_tot, m)
                val = jnp.stack([v0, v1], axis=1).reshape(s_tot, m)
            else:
                bitset = (idxp & jp) != 0
                pk = jnp.where(bitset, jnp.roll(key, jp, axis=0),
                               jnp.roll(key, -jp, axis=0))
                pv = jnp.where(bitset, jnp.roll(val, jp, axis=0),
                               jnp.roll(val, -jp, axis=0))
                le = key <= pk
                ge = ~(key < pk)
                if kp:
                    asc = (idxp & kp) == 0
                    want_min = jnp.logical_xor(bitset, asc)
                else:
                    want_min = ~bitset
                take_self = jnp.logical_or(
                    jnp.logical_and(want_min, le),
                    jnp.logical_and(~want_min, ge))
                key = jnp.where(take_self, key, pk)
                val = jnp.where(take_self, val, pv)
            j //= 2
        k *= 2

    LD = LD_ref[...]
    L, D = LD[:npad], LD[npad:]
    key_m = jnp.where(il < n, key, 0.0)
    cs, ts = [], []
    for bb in range(_BB):
        sl = slice(bb * npad, (bb + 1) * npad)
        cs.append(jnp.dot(L, val[sl], preferred_element_type=jnp.float32,
                          precision=jax.lax.Precision.HIGHEST))
        ts.append(jnp.dot(D, key_m[sl], preferred_element_type=jnp.float32,
                          precision=jax.lax.Precision.HIGHEST))
    c = jnp.concatenate(cs, axis=0)
    t = jnp.concatenate(ts, axis=0)

    prod = t * jnp.sin(c * pif_ref[...])
    acc = jnp.dot(sel_ref[...], prod, preferred_element_type=jnp.float32,
                  precision=jax.lax.Precision.HIGHEST)
    out_ref[0] = math.sqrt(2.0) * acc * ipif_ref[...]


def kernel(X, W, theta, freqs):
    b, n, d_in = X.shape
    m = theta.shape[0]
    npad = 1 << (n - 1).bit_length()
    assert npad == 256 and n == 200, "specialized to n=200 (npad=256)"
    s_tot = _BB * npad

    ilv = _sigma(np.arange(npad))
    src = np.minimum(ilv, n - 1)
    gidx = (np.arange(b)[:, None] * n + src[None, :]).reshape(-1)
    x2 = jnp.take(X.reshape(b * n, d_in), jnp.asarray(gidx), axis=0)

    wn = W / jnp.sum(W, axis=-1, keepdims=True)
    wpad = jnp.pad(wn, ((0, 0), (0, npad - n)))
    wcol = wpad[:, jnp.asarray(ilv)].reshape(b * npad, 1)

    th_t = theta.T
    Lm = (ilv[None, :] <= ilv[:, None]).astype(np.float32)
    Dm = np.eye(npad, dtype=np.float32) - \
        (ilv[None, :] == ilv[:, None] + 1).astype(np.float32)
    LD = jnp.asarray(np.concatenate([Lm, Dm], axis=0))

    seg = jnp.arange(s_tot, dtype=jnp.int32) // npad
    sel_m = (seg[None, :] == jnp.arange(_BB, dtype=jnp.int32)[:, None])
    sel_m = sel_m.astype(jnp.float32)
    pif = (jnp.pi * freqs)[None, :].astype(jnp.float32)
    ipif = (1.0 / (jnp.pi * freqs))[None, :].astype(jnp.float32)

    grid = b // _BB
    out = pl.pallas_call(
        _sw_body,
        grid=(grid,),
        in_specs=[
            pl.BlockSpec((s_tot, d_in), lambda i: (i, 0)),
            pl.BlockSpec((s_tot, 1), lambda i: (i, 0)),
            pl.BlockSpec((d_in, m), lambda i: (0, 0)),
            pl.BlockSpec((2 * npad, npad), lambda i: (0, 0)),
            pl.BlockSpec((_BB, s_tot), lambda i: (0, 0)),
            pl.BlockSpec((1, m), lambda i: (0, 0)),
            pl.BlockSpec((1, m), lambda i: (0, 0)),
        ],
        out_specs=pl.BlockSpec((1, _BB, m), lambda i: (i, 0, 0)),
        out_shape=jax.ShapeDtypeStruct((grid, _BB, m), jnp.float32),
    )(x2, wcol, th_t, LD, sel_m, pif, ipif)
    return out.reshape(b, m)
